# baseline (speedup 1.0000x reference)
import jax
import jax.numpy as jnp
from jax import lax
from jax.experimental import pallas as pl
from jax.experimental.pallas import tpu as pltpu

_BM = 256


def _body(L_ref, xt_ref, w1_ref, w2_ref, b_ref, o_ref, zt_ref, rt_ref):
    i = pl.program_id(0)

    @pl.when(i == 0)
    def _():
        zt_ref[...] = jnp.dot(
            w2_ref[...], xt_ref[...], preferred_element_type=jnp.float32
        ).astype(jnp.bfloat16)
        rt_ref[...] = (
            jnp.dot(w1_ref[...], xt_ref[...], preferred_element_type=jnp.float32)
            + b_ref[...]
        )

    acc = lax.dot_general(
        zt_ref[...],
        L_ref[...].astype(jnp.bfloat16),
        ((( 1,), (1,)), ((), ())),
        preferred_element_type=jnp.float32,
    )
    o_ref[...] = (acc + rt_ref[:, pl.ds(i * _BM, _BM)]).T


@jax.jit
def kernel(L, x, W, b):
    n, d = x.shape
    out = W.shape[0]
    w1 = W[:, :d]
    w2 = W[:, d:]
    xt = x.T
    b2 = b.reshape(out, 1)

    return pl.pallas_call(
        _body,
        grid=(n // _BM,),
        in_specs=[
            pl.BlockSpec((_BM, n), lambda i: (i, 0)),
            pl.BlockSpec((d, n), lambda i: (0, 0)),
            pl.BlockSpec((out, d), lambda i: (0, 0)),
            pl.BlockSpec((out, d), lambda i: (0, 0)),
            pl.BlockSpec((out, 1), lambda i: (0, 0)),
        ],
        out_specs=pl.BlockSpec((_BM, out), lambda i: (i, 0)),
        out_shape=jax.ShapeDtypeStruct((n, out), jnp.float32),
        scratch_shapes=[
            pltpu.VMEM((out, n), jnp.bfloat16),
            pltpu.VMEM((out, n), jnp.float32),
        ],
    )(L, xt, w1, w2, b2)
